# baseline (speedup 1.0000x reference)
import functools

import jax
import jax.numpy as jnp
from jax import lax
from jax.experimental import pallas as pl
from jax.experimental.pallas import tpu as pltpu
from jax.experimental.pallas import tpu_sc as plsc

NC = 2
NS = 16
NW = NC * NS
CHUNK = 128
NB = 10
LANES = 16


def _rsqrt16(x):
    i = plsc.bitcast(x, jnp.int32)
    i = jnp.int32(0x5F3759DF) - lax.shift_right_arithmetic(i, 1)
    y = plsc.bitcast(i, jnp.float32)
    xh = x * 0.5
    for _ in range(3):
        y = y * (1.5 - xh * y * y)
    return y


def _main_kernel(N, OUT, NCH, NPR):
    NP = NS * NPR
    TAIL = N - (NS - 1) * NPR
    M2 = -(-(-(-NCH // NS)) // 8) * 8
    F2 = NCH // M2
    REM2 = NCH - F2 * M2
    M1 = -(-(-(-NCH // NW)) // 8) * 8
    F1 = NCH // M1
    REM1 = NCH - F1 * M1
    mesh = plsc.VectorSubcoreMesh(core_axis_name="c", subcore_axis_name="s")

    NPO = NP * OUT // 128

    @functools.partial(
        pl.kernel,
        out_type=jax.ShapeDtypeStruct((NC, NPO, 128), jnp.float32),
        mesh=mesh,
        scratch_types=[
            pltpu.VMEM((M2, CHUNK), jnp.int32),
            pltpu.VMEM((M1, CHUNK), jnp.int32),
            pltpu.VMEM((M1, CHUNK), jnp.int32),
            pltpu.VMEM((CHUNK,), jnp.float32),
            pltpu.VMEM((NB, CHUNK, OUT), jnp.float32),
            pltpu.VMEM((NPR,), jnp.float32),
            pltpu.VMEM((NPR,), jnp.float32),
            pltpu.VMEM((NPR, OUT), jnp.float32),
            pltpu.VMEM((NPR, OUT), jnp.float32),
            pltpu.VMEM((NPR, OUT), jnp.float32),
            pltpu.VMEM((NPR * OUT // 128, 128), jnp.float32),
            pltpu.VMEM((OUT,), jnp.float32),
            pltpu.VMEM_SHARED((NP,), jnp.float32),
            pltpu.VMEM_SHARED((NP, OUT), jnp.float32),
            pltpu.VMEM_SHARED((NP, OUT), jnp.float32),
            [pltpu.SemaphoreType.DMA] * NB,
            [pltpu.SemaphoreType.DMA] * NB,
        ],
        compiler_params=pltpu.CompilerParams(
            use_tc_tiling_on_sc=False, needs_layout_passes=False
        ),
    )
    def main_k(h_hbm, ei_hbm, b_hbm, out_hbm, didx_v, msrc_v, mdst_v, ones_v,
               rows_v, degb, disb, hbuf, gslice, abuf, obuf, bbuf, deg_sp,
               g_sp, acc_sp, gsems, ssems):
        cid = lax.axis_index("c")
        sid = lax.axis_index("s")
        wid = cid * NS + sid
        base = sid * NPR

        ones16 = jnp.ones((LANES,), jnp.float32)
        zeros16 = jnp.zeros((LANES,), jnp.float32)
        for i in range(CHUNK // LANES):
            ones_v[pl.ds(i * LANES, LANES)] = ones16
        for i in range(NPR // LANES):
            degb[pl.ds(i * LANES, LANES)] = zeros16

        def emit_deg(cnt):
            G, T = cnt // NB, cnt % NB
            if G > 0:
                def agrp(i, c):
                    j0 = i * NB
                    for k in range(NB):
                        pltpu.async_copy(
                            ones_v, deg_sp.at[didx_v.at[j0 + k]], ssems[k],
                            add=True,
                        )

                    @pl.when(i > 0)
                    def _():
                        for k in range(NB):
                            pltpu.make_async_copy(
                                ones_v, deg_sp.at[didx_v.at[j0 - NB + k]],
                                ssems[k],
                            ).wait()

                    return c

                lax.fori_loop(0, G, agrp, 0)
                for k in range(NB):
                    pltpu.make_async_copy(
                        ones_v, deg_sp.at[didx_v.at[(G - 1) * NB + k]],
                        ssems[k],
                    ).wait()
            for t in range(T):
                pltpu.sync_copy(
                    ones_v, deg_sp.at[didx_v.at[G * NB + t]], add=True
                )

        if F2 > 0:
            @pl.when(sid < F2)
            def _():
                pltpu.sync_copy(ei_hbm.at[1, pl.ds(sid * M2, M2)], didx_v)

        if REM2 > 0:
            @pl.when(sid == F2)
            def _():
                pltpu.sync_copy(
                    ei_hbm.at[1, pl.ds(F2 * M2, REM2)],
                    didx_v.at[pl.ds(0, REM2)],
                )

        pltpu.sync_copy(degb, deg_sp.at[pl.ds(base, NPR)])
        plsc.subcore_barrier()

        if F2 > 0:
            @pl.when(sid < F2)
            def _():
                emit_deg(M2)

        if REM2 > 0:
            @pl.when(sid == F2)
            def _():
                emit_deg(REM2)

        if F1 > 0:
            @pl.when(wid < F1)
            def _():
                pltpu.sync_copy(ei_hbm.at[0, pl.ds(wid * M1, M1)], msrc_v)
                pltpu.sync_copy(ei_hbm.at[1, pl.ds(wid * M1, M1)], mdst_v)

        if REM1 > 0:
            @pl.when(wid == F1)
            def _():
                pltpu.sync_copy(
                    ei_hbm.at[0, pl.ds(F1 * M1, REM1)],
                    msrc_v.at[pl.ds(0, REM1)],
                )
                pltpu.sync_copy(
                    ei_hbm.at[1, pl.ds(F1 * M1, REM1)],
                    mdst_v.at[pl.ds(0, REM1)],
                )

        pltpu.sync_copy(b_hbm, bbuf)
        plsc.subcore_barrier()

        pltpu.sync_copy(deg_sp.at[pl.ds(base, NPR)], degb)

        @pl.when(sid < NS - 1)
        def _():
            pltpu.sync_copy(h_hbm.at[pl.ds(base, NPR)], hbuf)

        @pl.when(sid == NS - 1)
        def _():
            pltpu.sync_copy(
                h_hbm.at[pl.ds(base, TAIL)], hbuf.at[pl.ds(0, TAIL)]
            )
            for i in range(TAIL, NPR):
                hbuf[i, :] = zeros16

        def dis_loop(i, c):
            off = pl.multiple_of(i * LANES, LANES)
            d = degb[pl.ds(off, LANES)] + 1.0
            disb[pl.ds(off, LANES)] = _rsqrt16(d)
            return c

        lax.fori_loop(0, NPR // LANES, dis_loop, 0)

        def scale_loop(i, c):
            off = pl.multiple_of(i * LANES, LANES)
            dvec = disb[pl.ds(off, LANES)]
            for r in range(LANES):
                gslice[off + r, :] = hbuf[off + r, :] * dvec[r]
            return c

        lax.fori_loop(0, NPR // LANES, scale_loop, 0)
        pltpu.sync_copy(gslice, g_sp.at[pl.ds(base, NPR)])
        pltpu.sync_copy(gslice, acc_sp.at[pl.ds(base, NPR)])
        plsc.subcore_barrier()

        def emit_msg(cnt):
            G, T = cnt // NB, cnt % NB
            if G > 0:
                def mgrp(i, c):
                    j0 = i * NB

                    @pl.when(i > 0)
                    def _():
                        for k in range(NB):
                            pltpu.make_async_copy(
                                rows_v.at[k],
                                acc_sp.at[mdst_v.at[j0 - NB + k]],
                                ssems[k],
                            ).wait()

                    gd = [
                        pltpu.async_copy(
                            g_sp.at[msrc_v.at[j0 + k]], rows_v.at[k], gsems[k]
                        )
                        for k in range(NB)
                    ]
                    for k in range(NB):
                        gd[k].wait()
                        pltpu.async_copy(
                            rows_v.at[k], acc_sp.at[mdst_v.at[j0 + k]],
                            ssems[k], add=True,
                        )
                    return c

                lax.fori_loop(0, G, mgrp, 0)
                for k in range(NB):
                    pltpu.make_async_copy(
                        rows_v.at[k], acc_sp.at[mdst_v.at[(G - 1) * NB + k]],
                        ssems[k],
                    ).wait()
            for t in range(T):
                j = G * NB + t
                pltpu.async_copy(
                    g_sp.at[msrc_v.at[j]], rows_v.at[0], gsems[0]
                ).wait()
                pltpu.sync_copy(rows_v.at[0], acc_sp.at[mdst_v.at[j]], add=True)

        if F1 > 0:
            @pl.when(wid < F1)
            def _():
                emit_msg(M1)

        if REM1 > 0:
            @pl.when(wid == F1)
            def _():
                emit_msg(REM1)

        plsc.subcore_barrier()

        pltpu.sync_copy(acc_sp.at[pl.ds(base, NPR)], abuf)
        sel0 = lax.select(cid == 0, 1.0, 0.0)
        bvec = bbuf[...]

        ROWS_PER_128 = 128 // OUT

        def fin_loop(i, c):
            off = pl.multiple_of(i * LANES, LANES)
            dvec = disb[pl.ds(off, LANES)]
            for r in range(LANES):
                row = abuf[off + r, :] - (1.0 - sel0) * gslice[off + r, :]
                q = (LANES // ROWS_PER_128) * i + r // ROWS_PER_128
                obuf[q, pl.ds((r % ROWS_PER_128) * OUT, OUT)] = (
                    row * dvec[r] + sel0 * bvec
                )
            return c

        lax.fori_loop(0, NPR // LANES, fin_loop, 0)
        TPO = NPR * OUT // 128
        pltpu.sync_copy(obuf, out_hbm.at[cid, pl.ds(sid * TPO, TPO)])

    return main_k


def _pre_body(x_ref, w_ref, h_ref):
    h_ref[...] = jnp.dot(
        x_ref[...], w_ref[...], preferred_element_type=jnp.float32
    )


def _post_body(a_ref, o_ref):
    o_ref[...] = a_ref[0] + a_ref[1]


def kernel(x, edge_index, W, b):
    N, IN = x.shape
    OUT = W.shape[1]
    E = edge_index.shape[1]

    ei = edge_index.astype(jnp.int32)
    if E % CHUNK:
        pad = CHUNK - E % CHUNK
        ei = jnp.concatenate([ei, jnp.full((2, pad), N, jnp.int32)], axis=1)
    NCH = ei.shape[1] // CHUNK
    ei3 = ei.reshape(2, NCH, CHUNK)

    NPR = -(-(N + 1) // (NS * LANES)) * LANES
    NP = NS * NPR

    BLK = 2000 if N % 2000 == 0 else 8
    h = pl.pallas_call(
        _pre_body,
        grid=(N // BLK,),
        in_specs=[
            pl.BlockSpec((BLK, IN), lambda i: (i, 0)),
            pl.BlockSpec((IN, OUT), lambda i: (0, 0)),
        ],
        out_specs=pl.BlockSpec((BLK, OUT), lambda i: (i, 0)),
        out_shape=jax.ShapeDtypeStruct((N, OUT), jnp.float32),
    )(x, W)

    o = _main_kernel(N, OUT, NCH, NPR)(h, ei3, b)

    NPO = NP * OUT // 128
    PBLK = 256 if NPO % 256 == 0 else 8
    out128 = pl.pallas_call(
        _post_body,
        grid=(NPO // PBLK,),
        in_specs=[pl.BlockSpec((NC, PBLK, 128), lambda i: (0, i, 0))],
        out_specs=pl.BlockSpec((PBLK, 128), lambda i: (i, 0)),
        out_shape=jax.ShapeDtypeStruct((NPO, 128), jnp.float32),
    )(o)

    return (out128.reshape(NP, OUT)[:N], 0)
